# Initial kernel scaffold; baseline (speedup 1.0000x reference)
import jax
import jax.numpy as jnp
from jax.experimental import pallas as pl


def kernel(x, edge_index, edge_weight, W1, b1, W2, b2, W3, b3):
    raise NotImplementedError("write your pallas kernel here")


# speedup vs baseline: 5.2061x; 5.2061x over previous
import functools

import jax
import jax.numpy as jnp
from jax import lax
from jax.experimental import pallas as pl
from jax.experimental.pallas import tpu as pltpu
from jax.experimental.pallas import tpu_sc as plsc

N_NODES = 10000
N_PAD = 10240
D_FEAT = 128
NC = 2
NS = 16
LANES = 16
K_EDGES = 128
ROWS_PER_SUB = N_PAD // NS
ZROWS = K_EDGES


def _spmm_sc(rc, wts, dense, n_chunks):
    mesh = plsc.VectorSubcoreMesh(core_axis_name="c", subcore_axis_name="s")

    @functools.partial(
        pl.kernel,
        mesh=mesh,
        out_type=jax.ShapeDtypeStruct((NC, N_PAD, D_FEAT), jnp.float32),
        scratch_types=[
            pltpu.VMEM((2, 2, K_EDGES), jnp.int32),
            pltpu.VMEM((2, K_EDGES), jnp.float32),
            pltpu.VMEM((2, K_EDGES, D_FEAT), jnp.float32),
            pltpu.VMEM_SHARED((N_PAD, D_FEAT), jnp.float32),
            pltpu.SemaphoreType.DMA,
            pltpu.SemaphoreType.DMA,
        ],
    )
    def spmm(rc_hbm, w_hbm, dense_hbm, out_hbm, idx, wbuf, rows, acc, sem0, sem1):
        cid = lax.axis_index("c")
        sid = lax.axis_index("s")
        wid = cid * NS + sid
        sems = (sem0, sem1)

        def zero_body(r, _):
            for v in range(D_FEAT // LANES):
                rows[0, r, pl.ds(v * LANES, LANES)] = jnp.zeros(
                    (LANES,), jnp.float32
                )
            return 0

        lax.fori_loop(0, ZROWS, zero_body, 0)
        for t in range(ROWS_PER_SUB // ZROWS):
            base = sid * ROWS_PER_SUB + t * ZROWS
            pltpu.sync_copy(rows.at[0], acc.at[pl.ds(base, ZROWS)])
        plsc.subcore_barrier()

        def start(j, b):
            pltpu.sync_copy(rc_hbm.at[wid * n_chunks + j], idx.at[b])
            pltpu.sync_copy(w_hbm.at[wid * n_chunks + j], wbuf.at[b])
            pltpu.make_async_copy(
                dense_hbm.at[idx.at[b, 1]], rows.at[b], sems[b]
            ).start()

        start(0, 0)
        start(1, 1)

        def chunk_body(half, _):
            for b in range(2):
                j = half * 2 + b

                @pl.when(j < n_chunks)
                def _():
                    pltpu.make_async_copy(
                        dense_hbm.at[idx.at[b, 1]], rows.at[b], sems[b]
                    ).wait()

                    def scale_body(g, _c):
                        wv = wbuf[b, pl.ds(g * LANES, LANES)]
                        for l in range(LANES):
                            e = g * LANES + l
                            wvec = jnp.full((LANES,), wv[l], jnp.float32)
                            for v in range(D_FEAT // LANES):
                                sl = pl.ds(v * LANES, LANES)
                                rows[b, e, sl] = rows[b, e, sl] * wvec
                        return 0

                    lax.fori_loop(0, K_EDGES // LANES, scale_body, 0)
                    pltpu.sync_copy(rows.at[b], acc.at[idx.at[b, 0]], add=True)

                    @pl.when(j + 2 < n_chunks)
                    def _():
                        start(j + 2, b)

            return 0

        lax.fori_loop(0, (n_chunks + 1) // 2, chunk_body, 0)
        plsc.subcore_barrier()

        for t in range(ROWS_PER_SUB // ZROWS):
            base = sid * ROWS_PER_SUB + t * ZROWS
            pltpu.sync_copy(
                acc.at[pl.ds(base, ZROWS)], out_hbm.at[cid, pl.ds(base, ZROWS)]
            )

    return spmm(rc, wts, dense)


BN = 1000


def _mm_body(x_ref, w_ref, o_ref):
    o_ref[...] = jnp.dot(x_ref[...], w_ref[...], preferred_element_type=jnp.float32)


def _matmul(x, w):
    n, d = x.shape
    return pl.pallas_call(
        _mm_body,
        grid=(n // BN,),
        in_specs=[
            pl.BlockSpec((BN, d), lambda i: (i, 0)),
            pl.BlockSpec((d, w.shape[1]), lambda i: (0, 0)),
        ],
        out_specs=pl.BlockSpec((BN, w.shape[1]), lambda i: (i, 0)),
        out_shape=jax.ShapeDtypeStruct((n, w.shape[1]), jnp.float32),
    )(x, w)


def _fuse_relu_mm_body(p0_ref, p1_ref, b_ref, w_ref, o_ref):
    h = jax.nn.relu(p0_ref[...] + p1_ref[...] + b_ref[...])
    o_ref[...] = jnp.dot(h, w_ref[...], preferred_element_type=jnp.float32)


def _fuse_relu_mm(p0, p1, b, w):
    n, d = N_NODES, p0.shape[1]
    return pl.pallas_call(
        _fuse_relu_mm_body,
        grid=(n // BN,),
        in_specs=[
            pl.BlockSpec((BN, d), lambda i: (i, 0)),
            pl.BlockSpec((BN, d), lambda i: (i, 0)),
            pl.BlockSpec((1, d), lambda i: (0, 0)),
            pl.BlockSpec((d, w.shape[1]), lambda i: (0, 0)),
        ],
        out_specs=pl.BlockSpec((BN, w.shape[1]), lambda i: (i, 0)),
        out_shape=jax.ShapeDtypeStruct((n, w.shape[1]), jnp.float32),
    )(p0, p1, b, w)


def _mu_logvar_body(q0_ref, q1_ref, b_ref, mu_ref, lv_ref):
    t = q0_ref[...] + q1_ref[...] + b_ref[...]
    mu_ref[...] = t[:, : D_FEAT // 2]
    lv_ref[...] = t[:, D_FEAT // 2 :]


def _mu_logvar(q0, q1, bc):
    n, d = N_NODES, q0.shape[1]
    h = d // 2
    return pl.pallas_call(
        _mu_logvar_body,
        grid=(n // BN,),
        in_specs=[
            pl.BlockSpec((BN, d), lambda i: (i, 0)),
            pl.BlockSpec((BN, d), lambda i: (i, 0)),
            pl.BlockSpec((1, d), lambda i: (0, 0)),
        ],
        out_specs=[
            pl.BlockSpec((BN, h), lambda i: (i, 0)),
            pl.BlockSpec((BN, h), lambda i: (i, 0)),
        ],
        out_shape=[
            jax.ShapeDtypeStruct((n, h), jnp.float32),
            jax.ShapeDtypeStruct((n, h), jnp.float32),
        ],
    )(q0, q1, bc)


def _gram_body(a_ref, b_ref, o_ref):
    o_ref[...] = lax.dot_general(
        a_ref[...],
        b_ref[...],
        (((1,), (1,)), ((), ())),
        preferred_element_type=jnp.float32,
    )


BM_GRAM = 80


def _gram(z):
    n, d = z.shape
    return pl.pallas_call(
        _gram_body,
        grid=(n // BM_GRAM,),
        in_specs=[
            pl.BlockSpec((BM_GRAM, d), lambda i: (i, 0)),
            pl.BlockSpec((n, d), lambda i: (0, 0)),
        ],
        out_specs=pl.BlockSpec((BM_GRAM, n), lambda i: (i, 0)),
        out_shape=jax.ShapeDtypeStruct((n, n), jnp.float32),
    )(z, z)


def kernel(x, edge_index, edge_weight, W1, b1, W2, b2, W3, b3):
    row = edge_index[0].astype(jnp.int32)
    col = edge_index[1].astype(jnp.int32)
    w = edge_weight.astype(jnp.float32)

    e = row.shape[0]
    nw = NC * NS
    n_chunks = -(-e // (nw * K_EDGES))
    e_pad = nw * K_EDGES * n_chunks
    pad = e_pad - e
    if pad:
        row = jnp.concatenate([row, jnp.zeros((pad,), jnp.int32)])
        col = jnp.concatenate([col, jnp.zeros((pad,), jnp.int32)])
        w = jnp.concatenate([w, jnp.zeros((pad,), jnp.float32)])
    rc = jnp.stack(
        [
            row.reshape(nw * n_chunks, K_EDGES),
            col.reshape(nw * n_chunks, K_EDGES),
        ],
        axis=1,
    )
    wts = w.reshape(nw * n_chunks, K_EDGES)

    support1 = _matmul(x, W1)
    p = _spmm_sc(rc, wts, support1, n_chunks)
    wc = jnp.concatenate([W2, W3], axis=1)
    bc1 = b1.reshape(1, D_FEAT)
    hw = _fuse_relu_mm(p[0], p[1], bc1, wc)
    q = _spmm_sc(rc, wts, hw, n_chunks)
    bc23 = jnp.concatenate([b2, b3]).reshape(1, D_FEAT)
    mu, logvar = _mu_logvar(q[0], q[1], bc23)
    recon = _gram(mu)
    return (recon, mu, logvar, mu)
